# Initial kernel scaffold; baseline (speedup 1.0000x reference)
import jax
import jax.numpy as jnp
from jax.experimental import pallas as pl


def kernel(x, w1, w2, wf1, wf2, wf3, biases):
    raise NotImplementedError("write your pallas kernel here")


# speedup vs baseline: 1.0220x; 1.0220x over previous
import functools

import jax
import jax.numpy as jnp
from jax.experimental import pallas as pl
from jax.experimental.pallas import tpu as pltpu


def _fused_body(x_ref, w1_ref, w2_ref, wf1_ref, wf2_ref, wf3_ref,
                bias_ref, out_ref, *, bt):
    f32 = jnp.float32
    bf16 = jnp.bfloat16
    b1 = bias_ref[0:1, :]
    b2 = bias_ref[1:2, :]

    def c1(r):
        ka = (4 - r) * 128
        acc = jnp.dot(x_ref[0, : 7 * bt, r * 128:], w1_ref[:ka, :],
                      preferred_element_type=f32)
        return acc + jnp.dot(x_ref[0, bt:, : (r + 1) * 128], w1_ref[ka:, :],
                             preferred_element_type=f32)

    def pool1(a, b):
        m = jnp.maximum(a, b)
        return jnp.maximum(jnp.maximum(m[:, :128], m[:, 128:]) + b1, 0.0)

    p_even = pool1(c1(0), c1(1))
    p_odd = pool1(c1(2), c1(3))
    p1 = jnp.concatenate([p_even, p_odd], axis=1).astype(bf16)

    y2e = (jnp.dot(p1[: 5 * bt, :], w2_ref[:256, :], preferred_element_type=f32)
           + jnp.dot(p1[bt:6 * bt, :], w2_ref[256:512, :], preferred_element_type=f32)
           + jnp.dot(p1[2 * bt:, :128], w2_ref[512:, :], preferred_element_type=f32))
    y2o = (jnp.dot(p1[: 5 * bt, 128:], w2_ref[:128, :], preferred_element_type=f32)
           + jnp.dot(p1[bt:6 * bt, :], w2_ref[128:384, :], preferred_element_type=f32)
           + jnp.dot(p1[2 * bt:, :], w2_ref[384:, :], preferred_element_type=f32))
    m2 = jnp.maximum(y2e, y2o)
    p2 = jnp.maximum(jnp.maximum(m2[:, :128], m2[:, 128:]) + b2, 0.0).astype(bf16)

    f1in = jnp.concatenate([p2[q * bt:(q + 1) * bt, :] for q in range(5)], axis=1)
    f1 = jnp.dot(f1in, wf1_ref[...], preferred_element_type=f32) + bias_ref[2:3, :]
    f1 = jnp.maximum(f1, 0.0).astype(bf16)
    f2 = jnp.dot(f1, wf2_ref[...], preferred_element_type=f32) + bias_ref[3:4, :]
    f2 = jnp.maximum(f2, 0.0).astype(bf16)
    f3 = jnp.dot(f2, wf3_ref[...], preferred_element_type=f32) + bias_ref[4:5, :]
    out_ref[...] = f3.astype(out_ref.dtype)


def kernel(x, w1, w2, wf1, wf2, wf3, biases):
    N = x.shape[0]
    bt = 256
    while bt > 8 and (N + bt - 1) // bt < 2:
        bt //= 2
    n_pad = (-N) % bt
    nt = (N + n_pad) // bt

    xb = x.astype(jnp.bfloat16)
    if n_pad:
        xb = jnp.pad(xb, ((0, n_pad), (0, 0), (0, 0), (0, 0)))
    xb = jnp.transpose(xb, (0, 2, 3, 1))
    xb = xb.reshape(nt, bt, 8, 4, 96)
    xb = jnp.pad(xb, ((0, 0), (0, 0), (0, 0), (0, 0), (0, 32)))
    xb = jnp.transpose(xb, (0, 2, 1, 3, 4)).reshape(nt, 8 * bt, 512)

    out = pl.pallas_call(
        functools.partial(_fused_body, bt=bt),
        out_shape=jax.ShapeDtypeStruct((nt * bt, 128), jnp.float32),
        grid=(nt,),
        in_specs=[
            pl.BlockSpec((1, 8 * bt, 512), lambda i: (i, 0, 0)),
            pl.BlockSpec((640, 256), lambda i: (0, 0)),
            pl.BlockSpec((640, 256), lambda i: (0, 0)),
            pl.BlockSpec((640, 128), lambda i: (0, 0)),
            pl.BlockSpec((128, 128), lambda i: (0, 0)),
            pl.BlockSpec((128, 128), lambda i: (0, 0)),
            pl.BlockSpec((8, 128), lambda i: (0, 0)),
        ],
        out_specs=pl.BlockSpec((bt, 128), lambda i: (i, 0)),
        compiler_params=pltpu.CompilerParams(
            dimension_semantics=("parallel",),
            vmem_limit_bytes=64 * 1024 * 1024,
            allow_input_fusion=[True] + [False] * 6,
        ),
    )(xb, w1, w2, wf1, wf2, wf3, biases)

    return out[:N, :10]
